# Initial kernel scaffold; baseline (speedup 1.0000x reference)
import jax
import jax.numpy as jnp
from jax.experimental import pallas as pl


def kernel(x, edge_index, W1, b1, W2, b2):
    raise NotImplementedError("write your pallas kernel here")


# speedup vs baseline: 2.8952x; 2.8952x over previous
import functools

import jax
import jax.numpy as jnp
from jax import lax
from jax.experimental import pallas as pl
from jax.experimental.pallas import tpu as pltpu
from jax.experimental.pallas import tpu_sc as plsc

N = 10000
NPAD = 10240
E = 320000
D = 128
NC = 2
NS = 16
NW = NC * NS
CHUNK = 128
NCHUNK = 80
EPAD = NW * NCHUNK * CHUNK
RPT = NPAD // NS
DEGW = 16
RB = 400
GRID = N // RB


def _sc_mesh():
    return plsc.VectorSubcoreMesh(core_axis_name="c", subcore_axis_name="s")


@functools.partial(
    pl.kernel,
    out_type=jax.ShapeDtypeStruct((NC * 2 * NPAD,), jnp.float32),
    mesh=_sc_mesh(),
    scratch_types=[
        pltpu.VMEM((CHUNK,), jnp.float32),
        pltpu.VMEM((RPT,), jnp.float32),
        pltpu.VMEM((CHUNK,), jnp.int32),
        pltpu.VMEM((CHUNK,), jnp.int32),
        pltpu.VMEM_SHARED((NPAD,), jnp.float32),
        pltpu.VMEM_SHARED((NPAD,), jnp.float32),
    ],
)
def _degree_kernel(src_hbm, dst_hbm, out_hbm,
                   ones_v, stage, icur_s, icur_d, acc_a, acc_b):
    c = lax.axis_index("c")
    s = lax.axis_index("s")
    wid = s * NC + c
    row0 = s * RPT
    base = wid * NCHUNK * CHUNK
    lanes = lax.iota(jnp.int32, 16)
    one16 = jnp.where(lanes >= 0, 1.0, 0.0)
    zero16 = jnp.where(lanes < 0, 1.0, 0.0)

    for q in range(CHUNK // 16):
        ones_v[pl.ds(q * 16, 16)] = one16

    def fill(i, carry):
        stage[pl.ds(i * 16, 16)] = zero16
        return carry

    lax.fori_loop(0, RPT // 16, fill, 0)
    pltpu.sync_copy(stage, acc_a.at[pl.ds(row0, RPT)])
    pltpu.sync_copy(stage, acc_b.at[pl.ds(row0, RPT)])
    plsc.subcore_barrier()

    def body(j, carry):
        off = base + j * CHUNK
        pltpu.sync_copy(src_hbm.at[pl.ds(off, CHUNK)], icur_s)
        pltpu.sync_copy(ones_v, acc_a.at[icur_s], add=True)
        pltpu.sync_copy(dst_hbm.at[pl.ds(off, CHUNK)], icur_d)
        pltpu.sync_copy(ones_v, acc_b.at[icur_d], add=True)
        return carry

    lax.fori_loop(0, NCHUNK, body, 0)
    plsc.subcore_barrier()
    pltpu.sync_copy(acc_a.at[pl.ds(row0, RPT)], stage)
    pltpu.sync_copy(stage, out_hbm.at[pl.ds(c * 2 * NPAD + row0, RPT)])
    pltpu.sync_copy(acc_b.at[pl.ds(row0, RPT)], stage)
    pltpu.sync_copy(stage, out_hbm.at[pl.ds(c * 2 * NPAD + NPAD + row0, RPT)])


@functools.partial(
    pl.kernel,
    out_type=jax.ShapeDtypeStruct((NC * NPAD, D), jnp.float32),
    mesh=_sc_mesh(),
    scratch_types=[
        pltpu.VMEM((CHUNK, D), jnp.float32),
        pltpu.VMEM((CHUNK,), jnp.int32),
        pltpu.VMEM((CHUNK,), jnp.int32),
        pltpu.VMEM_SHARED((NPAD, D), jnp.float32),
        pltpu.SemaphoreType.DMA,
    ],
)
def _edge_kernel(h_hbm, src_hbm, dst_hbm, out_hbm,
                 rows, icur_s, icur_d, acc, sem):
    c = lax.axis_index("c")
    s = lax.axis_index("s")
    wid = s * NC + c
    row0 = s * RPT
    lanes = lax.iota(jnp.int32, 16)
    zero16 = jnp.where(lanes < 0, 1.0, 0.0)

    def fill(i, carry):
        r = i // 8
        q = (i % 8) * 16
        rows[r, pl.ds(q, 16)] = zero16
        return carry

    lax.fori_loop(0, CHUNK * 8, fill, 0)
    for k in range(RPT // CHUNK):
        pltpu.sync_copy(rows, acc.at[pl.ds(row0 + k * CHUNK, CHUNK)])
    plsc.subcore_barrier()
    base = wid * NCHUNK * CHUNK

    def body(j, carry):
        off = base + j * CHUNK
        pltpu.sync_copy(src_hbm.at[pl.ds(off, CHUNK)], icur_s)
        pltpu.async_copy(h_hbm.at[icur_s], rows, sem).wait()
        pltpu.sync_copy(dst_hbm.at[pl.ds(off, CHUNK)], icur_d)
        pltpu.sync_copy(rows, acc.at[icur_d], add=True)
        return carry

    lax.fori_loop(0, NCHUNK, body, 0)
    plsc.subcore_barrier()
    for k in range(RPT // CHUNK):
        pltpu.sync_copy(acc.at[pl.ds(row0 + k * CHUNK, CHUNK)], rows)
        pltpu.sync_copy(rows,
                        out_hbm.at[pl.ds(c * NPAD + row0 + k * CHUNK, CHUNK)])


def _tc1_body(degp_ref, x_ref, w_ref, h_ref, ns_ref, nd_ref):
    p = degp_ref[...]
    deg_o = p[:, 0:1] + p[:, 2:3]
    deg_i = p[:, 1:2] + p[:, 3:4]
    ns = lax.rsqrt(jnp.where(deg_o > 0.0, deg_o, 1.0))
    nd = lax.rsqrt(jnp.where(deg_i > 0.0, deg_i, 1.0))
    ns_ref[...] = ns
    nd_ref[...] = nd
    h_ref[...] = jnp.dot(x_ref[...] * ns, w_ref[...],
                         preferred_element_type=jnp.float32)


_tc1 = pl.pallas_call(
    _tc1_body,
    grid=(GRID,),
    in_specs=[
        pl.BlockSpec((RB, 4), lambda i: (i, 0)),
        pl.BlockSpec((RB, D), lambda i: (i, 0)),
        pl.BlockSpec((D, D), lambda i: (0, 0)),
    ],
    out_specs=[
        pl.BlockSpec((RB, D), lambda i: (i, 0)),
        pl.BlockSpec((RB, 1), lambda i: (i, 0)),
        pl.BlockSpec((RB, 1), lambda i: (i, 0)),
    ],
    out_shape=[
        jax.ShapeDtypeStruct((NPAD, D), jnp.float32),
        jax.ShapeDtypeStruct((N, 1), jnp.float32),
        jax.ShapeDtypeStruct((N, 1), jnp.float32),
    ],
)


def _tc2_body(part_ref, ns_ref, nd_ref, b_ref, w_ref, o_ref):
    agg = part_ref[0] + part_ref[1]
    h = jnp.maximum(agg * nd_ref[...] + b_ref[...], 0.0)
    o_ref[...] = jnp.dot(h * ns_ref[...], w_ref[...],
                         preferred_element_type=jnp.float32)


_tc2 = pl.pallas_call(
    _tc2_body,
    grid=(GRID,),
    in_specs=[
        pl.BlockSpec((NC, RB, D), lambda i: (0, i, 0)),
        pl.BlockSpec((RB, 1), lambda i: (i, 0)),
        pl.BlockSpec((RB, 1), lambda i: (i, 0)),
        pl.BlockSpec((1, D), lambda i: (0, 0)),
        pl.BlockSpec((D, D), lambda i: (0, 0)),
    ],
    out_specs=pl.BlockSpec((RB, D), lambda i: (i, 0)),
    out_shape=jax.ShapeDtypeStruct((NPAD, D), jnp.float32),
)


def _tc3_body(part_ref, nd_ref, b_ref, o_ref):
    agg = part_ref[0] + part_ref[1]
    o_ref[...] = agg * nd_ref[...] + b_ref[...]


_tc3 = pl.pallas_call(
    _tc3_body,
    grid=(GRID,),
    in_specs=[
        pl.BlockSpec((NC, RB, D), lambda i: (0, i, 0)),
        pl.BlockSpec((RB, 1), lambda i: (i, 0)),
        pl.BlockSpec((1, D), lambda i: (0, 0)),
    ],
    out_specs=pl.BlockSpec((RB, D), lambda i: (i, 0)),
    out_shape=jax.ShapeDtypeStruct((N, D), jnp.float32),
)


def kernel(x, edge_index, W1, b1, W2, b2):
    pad = jnp.full((EPAD - E,), N, dtype=jnp.int32)
    src = jnp.concatenate([edge_index[0], pad])
    dst = jnp.concatenate([edge_index[1], pad])
    degt = _degree_kernel(src, dst).reshape(NC * 2, NPAD).T
    h1, ns, nd = _tc1(degt, x, W1)
    p1 = _edge_kernel(h1, src, dst).reshape(NC, NPAD, D)
    h2 = _tc2(p1, ns, nd, b1[None, :], W2)
    p2 = _edge_kernel(h2, src, dst).reshape(NC, NPAD, D)
    out = _tc3(p2, nd, b2[None, :])
    return out


# speedup vs baseline: 3.5355x; 1.2212x over previous
import functools

import jax
import jax.numpy as jnp
from jax import lax
from jax.experimental import pallas as pl
from jax.experimental.pallas import tpu as pltpu
from jax.experimental.pallas import tpu_sc as plsc

N = 10000
NPAD = 10240
E = 320000
D = 128
NC = 2
NS = 16
NW = NC * NS
CHUNK = 128
NCHUNK = 80
EPAD = NW * NCHUNK * CHUNK
RPT = NPAD // NS
DEGW = 16
RB = 400
GRID = N // RB


def _sc_mesh():
    return plsc.VectorSubcoreMesh(core_axis_name="c", subcore_axis_name="s")


@functools.partial(
    pl.kernel,
    out_type=jax.ShapeDtypeStruct((NC * 2 * NPAD,), jnp.float32),
    mesh=_sc_mesh(),
    scratch_types=[
        pltpu.VMEM((CHUNK,), jnp.float32),
        pltpu.VMEM((RPT,), jnp.float32),
        pltpu.VMEM((CHUNK,), jnp.int32),
        pltpu.VMEM((CHUNK,), jnp.int32),
        pltpu.VMEM_SHARED((NPAD,), jnp.float32),
        pltpu.VMEM_SHARED((NPAD,), jnp.float32),
    ],
)
def _degree_kernel(src_hbm, dst_hbm, out_hbm,
                   ones_v, stage, icur_s, icur_d, acc_a, acc_b):
    c = lax.axis_index("c")
    s = lax.axis_index("s")
    wid = s * NC + c
    row0 = s * RPT
    base = wid * NCHUNK * CHUNK
    lanes = lax.iota(jnp.int32, 16)
    one16 = jnp.where(lanes >= 0, 1.0, 0.0)
    zero16 = jnp.where(lanes < 0, 1.0, 0.0)

    for q in range(CHUNK // 16):
        ones_v[pl.ds(q * 16, 16)] = one16

    def fill(i, carry):
        stage[pl.ds(i * 16, 16)] = zero16
        return carry

    lax.fori_loop(0, RPT // 16, fill, 0)
    pltpu.sync_copy(stage, acc_a.at[pl.ds(row0, RPT)])
    pltpu.sync_copy(stage, acc_b.at[pl.ds(row0, RPT)])
    plsc.subcore_barrier()

    def body(j, carry):
        off = base + j * CHUNK
        pltpu.sync_copy(src_hbm.at[pl.ds(off, CHUNK)], icur_s)
        pltpu.sync_copy(ones_v, acc_a.at[icur_s], add=True)
        pltpu.sync_copy(dst_hbm.at[pl.ds(off, CHUNK)], icur_d)
        pltpu.sync_copy(ones_v, acc_b.at[icur_d], add=True)
        return carry

    lax.fori_loop(0, NCHUNK, body, 0)
    plsc.subcore_barrier()
    pltpu.sync_copy(acc_a.at[pl.ds(row0, RPT)], stage)
    pltpu.sync_copy(stage, out_hbm.at[pl.ds(c * 2 * NPAD + row0, RPT)])
    pltpu.sync_copy(acc_b.at[pl.ds(row0, RPT)], stage)
    pltpu.sync_copy(stage, out_hbm.at[pl.ds(c * 2 * NPAD + NPAD + row0, RPT)])


NBUF = 2
NGRP = NCHUNK // NBUF
EPWP = NCHUNK * CHUNK


@functools.partial(
    pl.kernel,
    out_type=jax.ShapeDtypeStruct((NC * NPAD, D), jnp.float32),
    mesh=_sc_mesh(),
    scratch_types=[
        pltpu.VMEM((CHUNK, D), jnp.float32),
        pltpu.VMEM((CHUNK, D), jnp.float32),
        pltpu.VMEM((CHUNK,), jnp.int32),
        pltpu.VMEM((CHUNK,), jnp.int32),
        pltpu.VMEM((CHUNK,), jnp.int32),
        pltpu.VMEM((CHUNK,), jnp.int32),
        pltpu.VMEM_SHARED((NPAD, D), jnp.float32),
        pltpu.SemaphoreType.DMA,
        pltpu.SemaphoreType.DMA,
    ],
)
def _edge_kernel(h_hbm, src_hbm, dst_hbm, out_hbm,
                 rows0, rows1, is0, is1, id0, id1,
                 acc_edge, gs0, gs1):
    rows = (rows0, rows1)
    isb = (is0, is1)
    idb = (id0, id1)
    gsem = (gs0, gs1)
    c = lax.axis_index("c")
    s = lax.axis_index("s")
    wid = s * NC + c
    row0 = s * RPT
    base = wid * EPWP
    lanes = lax.iota(jnp.int32, 16)
    zero16 = jnp.where(lanes < 0, 1.0, 0.0)

    def fill(i, carry):
        r = i // 8
        q = (i % 8) * 16
        rows0[r, pl.ds(q, 16)] = zero16
        return carry

    lax.fori_loop(0, CHUNK * 8, fill, 0)
    for k in range(RPT // CHUNK):
        pltpu.sync_copy(rows0, acc_edge.at[pl.ds(row0 + k * CHUNK, CHUNK)])
    plsc.subcore_barrier()

    def load_idx(j, b):
        off = base + j * CHUNK
        pltpu.sync_copy(src_hbm.at[pl.ds(off, CHUNK)], isb[b])
        pltpu.sync_copy(dst_hbm.at[pl.ds(off, CHUNK)], idb[b])

    def gather_start(b):
        pltpu.async_copy(h_hbm.at[isb[b]], rows[b], gsem[b])

    def consume(b):
        pltpu.make_async_copy(
            h_hbm.at[pl.ds(0, CHUNK)], rows[b], gsem[b]).wait()
        pltpu.sync_copy(rows[b], acc_edge.at[idb[b]], add=True)

    for b in range(NBUF):
        load_idx(b, b)
        gather_start(b)

    def group(g, carry):
        for b in range(NBUF):
            consume(b)
            load_idx(g * NBUF + b + NBUF, b)
            gather_start(b)
        return carry

    lax.fori_loop(0, NGRP - 1, group, 0)
    for b in range(NBUF):
        consume(b)
    plsc.subcore_barrier()
    for k in range(RPT // CHUNK):
        pltpu.sync_copy(acc_edge.at[pl.ds(row0 + k * CHUNK, CHUNK)], rows0)
        pltpu.sync_copy(rows0,
                        out_hbm.at[pl.ds(c * NPAD + row0 + k * CHUNK, CHUNK)])


def _tc1_body(degp_ref, x_ref, w_ref, h_ref, ns_ref, nd_ref):
    p = degp_ref[...]
    deg_o = p[:, 0:1] + p[:, 2:3]
    deg_i = p[:, 1:2] + p[:, 3:4]
    ns = lax.rsqrt(jnp.where(deg_o > 0.0, deg_o, 1.0))
    nd = lax.rsqrt(jnp.where(deg_i > 0.0, deg_i, 1.0))
    ns_ref[...] = ns
    nd_ref[...] = nd
    h_ref[...] = jnp.dot(x_ref[...] * ns, w_ref[...],
                         preferred_element_type=jnp.float32)


_tc1 = pl.pallas_call(
    _tc1_body,
    grid=(GRID,),
    in_specs=[
        pl.BlockSpec((RB, 4), lambda i: (i, 0)),
        pl.BlockSpec((RB, D), lambda i: (i, 0)),
        pl.BlockSpec((D, D), lambda i: (0, 0)),
    ],
    out_specs=[
        pl.BlockSpec((RB, D), lambda i: (i, 0)),
        pl.BlockSpec((RB, 1), lambda i: (i, 0)),
        pl.BlockSpec((RB, 1), lambda i: (i, 0)),
    ],
    out_shape=[
        jax.ShapeDtypeStruct((NPAD, D), jnp.float32),
        jax.ShapeDtypeStruct((N, 1), jnp.float32),
        jax.ShapeDtypeStruct((N, 1), jnp.float32),
    ],
)


def _tc2_body(part_ref, ns_ref, nd_ref, b_ref, w_ref, o_ref):
    agg = part_ref[0] + part_ref[1]
    h = jnp.maximum(agg * nd_ref[...] + b_ref[...], 0.0)
    o_ref[...] = jnp.dot(h * ns_ref[...], w_ref[...],
                         preferred_element_type=jnp.float32)


_tc2 = pl.pallas_call(
    _tc2_body,
    grid=(GRID,),
    in_specs=[
        pl.BlockSpec((NC, RB, D), lambda i: (0, i, 0)),
        pl.BlockSpec((RB, 1), lambda i: (i, 0)),
        pl.BlockSpec((RB, 1), lambda i: (i, 0)),
        pl.BlockSpec((1, D), lambda i: (0, 0)),
        pl.BlockSpec((D, D), lambda i: (0, 0)),
    ],
    out_specs=pl.BlockSpec((RB, D), lambda i: (i, 0)),
    out_shape=jax.ShapeDtypeStruct((NPAD, D), jnp.float32),
)


def _tc3_body(part_ref, nd_ref, b_ref, o_ref):
    agg = part_ref[0] + part_ref[1]
    o_ref[...] = agg * nd_ref[...] + b_ref[...]


_tc3 = pl.pallas_call(
    _tc3_body,
    grid=(GRID,),
    in_specs=[
        pl.BlockSpec((NC, RB, D), lambda i: (0, i, 0)),
        pl.BlockSpec((RB, 1), lambda i: (i, 0)),
        pl.BlockSpec((1, D), lambda i: (0, 0)),
    ],
    out_specs=pl.BlockSpec((RB, D), lambda i: (i, 0)),
    out_shape=jax.ShapeDtypeStruct((N, D), jnp.float32),
)


def kernel(x, edge_index, W1, b1, W2, b2):
    pad = jnp.full((EPAD - E,), N, dtype=jnp.int32)
    src = jnp.concatenate([edge_index[0], pad])
    dst = jnp.concatenate([edge_index[1], pad])
    degt = _degree_kernel(src, dst).reshape(NC * 2, NPAD).T
    h1, ns, nd = _tc1(degt, x, W1)
    p1 = _edge_kernel(h1, src, dst).reshape(NC, NPAD, D)
    h2 = _tc2(p1, ns, nd, b1[None, :], W2)
    p2 = _edge_kernel(h2, src, dst).reshape(NC, NPAD, D)
    out = _tc3(p2, nd, b2[None, :])
    return out
